# baseline (speedup 1.0000x reference)
import functools

import jax
import jax.numpy as jnp
from jax import lax
from jax.experimental import pallas as pl
from jax.experimental.pallas import tpu as pltpu
from jax.experimental.pallas import tpu_sc as plsc

_ROWS = 16384
_COLS = 4096
_PERIOD = 4096

_NW = 32
_ROWS_PER_W = _ROWS // _NW
_CHUNK = 16
_NCH = _ROWS_PER_W // _CHUNK

_mesh = plsc.VectorSubcoreMesh(core_axis_name="c", subcore_axis_name="s")


@functools.partial(
    pl.kernel,
    mesh=_mesh,
    out_type=jax.ShapeDtypeStruct((_ROWS, _COLS), jnp.float32),
    scratch_types=[
        pltpu.VMEM((_CHUNK, _COLS), jnp.float32),
    ],
)
def _sc_copy(x_hbm, o_hbm, buf):
    wid = lax.axis_index("s") * 2 + lax.axis_index("c")
    base = wid * _ROWS_PER_W

    def body(i, carry):
        pltpu.sync_copy(x_hbm.at[pl.ds(base + i * _CHUNK, _CHUNK)], buf)
        pltpu.sync_copy(buf, o_hbm.at[pl.ds(base + i * _CHUNK, _CHUNK)])
        return carry

    lax.fori_loop(0, _NCH, body, 0)


def kernel(x):
    out = _sc_copy(x)
    return jnp.reshape(out, (_ROWS // _PERIOD, _PERIOD, _COLS))


# speedup vs baseline: 1.0535x; 1.0535x over previous
import functools

import jax
import jax.numpy as jnp
from jax import lax
from jax.experimental import pallas as pl
from jax.experimental.pallas import tpu as pltpu
from jax.experimental.pallas import tpu_sc as plsc

_ROWS = 16384
_COLS = 4096
_PERIOD = 4096

_NW = 32
_ROWS_PER_W = _ROWS // _NW
_CHUNK = 8
_NCH = _ROWS_PER_W // _CHUNK
_NBUF = 3

_mesh = plsc.VectorSubcoreMesh(core_axis_name="c", subcore_axis_name="s")


@functools.partial(
    pl.kernel,
    mesh=_mesh,
    out_type=jax.ShapeDtypeStruct((_ROWS, _COLS), jnp.float32),
    scratch_types=(
        [pltpu.VMEM((_CHUNK, _COLS), jnp.float32) for _ in range(_NBUF)]
        + [pltpu.SemaphoreType.DMA for _ in range(2 * _NBUF)]
    ),
)
def _sc_copy(x_hbm, o_hbm, *scratch):
    bufs = scratch[:_NBUF]
    isems = scratch[_NBUF:2 * _NBUF]
    osems = scratch[2 * _NBUF:]
    wid = lax.axis_index("s") * 2 + lax.axis_index("c")
    base = wid * _ROWS_PER_W

    def in_copy(i):
        b = i % _NBUF
        return pltpu.make_async_copy(
            x_hbm.at[pl.ds(base + i * _CHUNK, _CHUNK)],
            bufs[b],
            isems[b],
        )

    def out_copy(i):
        b = i % _NBUF
        return pltpu.make_async_copy(
            bufs[b],
            o_hbm.at[pl.ds(base + i * _CHUNK, _CHUNK)],
            osems[b],
        )

    for s in range(_NBUF):
        in_copy(s).start()
    for i in range(_NCH):
        in_copy(i).wait()
        out_copy(i).start()
        oldest = i - (_NBUF - 1)
        if oldest >= 0 and oldest + _NBUF < _NCH:
            out_copy(oldest).wait()
            in_copy(oldest + _NBUF).start()
    for i in range(max(_NCH - _NBUF, 0), _NCH):
        out_copy(i).wait()


def kernel(x):
    out = _sc_copy(x)
    return jnp.reshape(out, (_ROWS // _PERIOD, _PERIOD, _COLS))
